# baseline (speedup 1.0000x reference)
import functools

import jax
import jax.numpy as jnp
from jax import lax
from jax.experimental import pallas as pl
from jax.experimental.pallas import tpu as pltpu
from jax.experimental.pallas import tpu_sc as plsc

N = 10000
E = 320000
D = 128
DE = 16

NP = 10240
NC = 2
NS = 16
NW = NC * NS
EW = E // NW
CHUNK = 80
NCHUNK = EW // CHUNK
BLK = 5
NBLK = NCHUNK // BLK
ROWS_W = NP // NS

_SC_PARAMS = pltpu.CompilerParams(
    needs_layout_passes=False, use_tc_tiling_on_sc=False)


def _tc1_body(x_ref, w_ref, asd_ref, ft_ref, sd_ref):
    xb = x_ref[...]
    ft_ref[...] = jnp.dot(xb, w_ref[...], preferred_element_type=jnp.float32)
    sd_ref[...] = jnp.dot(xb, asd_ref[...], preferred_element_type=jnp.float32)


def _tc1(x, W, asd):
    grid = 10
    nb = N // grid
    return pl.pallas_call(
        _tc1_body,
        grid=(grid,),
        in_specs=[
            pl.BlockSpec((nb, D), lambda i: (i, 0)),
            pl.BlockSpec((D, D), lambda i: (0, 0)),
            pl.BlockSpec((D, 2), lambda i: (0, 0)),
        ],
        out_specs=[
            pl.BlockSpec((nb, D), lambda i: (i, 0)),
            pl.BlockSpec((nb, 2), lambda i: (i, 0)),
        ],
        out_shape=[
            jax.ShapeDtypeStruct((N, D), jnp.float32),
            jax.ShapeDtypeStruct((N, 2), jnp.float32),
        ],
    )(x, W, asd)


def _tc1e_body(ae_ref, eattr_ref, ea_ref):
    @pl.when(pl.program_id(0) == 0)
    def _():
        ea_ref[...] = jnp.zeros_like(ea_ref)
    ea_ref[...] += jnp.dot(eattr_ref[...], ae_ref[...],
                           preferred_element_type=jnp.float32)


def _tc1e(ae_blkdiag, edge_attr128):
    grid = 8
    kb = 128 * DE // grid
    return pl.pallas_call(
        _tc1e_body,
        grid=(grid,),
        in_specs=[
            pl.BlockSpec((kb, 128), lambda i: (i, 0)),
            pl.BlockSpec((E // 128, kb), lambda i: (0, i)),
        ],
        out_specs=pl.BlockSpec((E // 128, 128), lambda i: (0, 0)),
        out_shape=jax.ShapeDtypeStruct((E // 128, 128), jnp.float32),
    )(ae_blkdiag, edge_attr128)


def _sc_scalar_pass(sdt, srcr, dstr, ear):
    mesh = plsc.VectorSubcoreMesh(core_axis_name="c", subcore_axis_name="s")

    @functools.partial(
        pl.kernel,
        mesh=mesh,
        compiler_params=_SC_PARAMS,
        out_type=[
            jax.ShapeDtypeStruct((NW, NCHUNK, CHUNK), jnp.float32),
            jax.ShapeDtypeStruct((NC, NP), jnp.float32),
        ],
        scratch_types=[
            pltpu.VMEM((2 * NP,), jnp.float32),
            pltpu.VMEM((NCHUNK, CHUNK), jnp.int32),
            pltpu.VMEM((NCHUNK, CHUNK), jnp.int32),
            pltpu.VMEM((NCHUNK, CHUNK), jnp.float32),
            pltpu.VMEM((NP,), jnp.float32),
            pltpu.VMEM((NS, ROWS_W), jnp.float32),
            pltpu.VMEM((ROWS_W,), jnp.float32),
            pltpu.VMEM_SHARED((NS, NP), jnp.float32),
        ],
    )
    def sc_a(sdt_hbm, src_hbm, dst_hbm, ea_hbm, e_hbm, ss_hbm,
             sdt_v, srcc, dstc, eac, ss_tab, ssg, ssred, ss_all):
        cid = lax.axis_index("c")
        tid = lax.axis_index("s")
        wid = tid * NC + cid
        r0 = tid * ROWS_W

        zf16 = jnp.zeros((16,), jnp.float32)

        pltpu.sync_copy(sdt_hbm, sdt_v)
        pltpu.sync_copy(src_hbm.at[wid], srcc)
        pltpu.sync_copy(dst_hbm.at[wid], dstc)
        pltpu.sync_copy(ea_hbm.at[wid], eac)

        def zss(i, _):
            ss_tab[pl.ds(i * 16, 16)] = zf16
            return 0
        lax.fori_loop(0, NP // 16, zss, 0)

        def edge_chunk(c, _):
            def edge_grp(g, _):
                srcv = srcc[c, pl.ds(g * 16, 16)]
                dstv = dstc[c, pl.ds(g * 16, 16)]
                eav = eac[c, pl.ds(g * 16, 16)]
                s = plsc.load_gather(sdt_v, [srcv * 2])
                dd = plsc.load_gather(sdt_v, [dstv * 2 + 1])
                ev = jnp.exp(s + dd + eav)
                eac[c, pl.ds(g * 16, 16)] = ev
                plsc.addupdate_scatter(ss_tab, [dstv], ev)
                return 0
            lax.fori_loop(0, CHUNK // 16, edge_grp, 0)
            return 0
        lax.fori_loop(0, NCHUNK, edge_chunk, 0)

        pltpu.sync_copy(eac, e_hbm.at[wid])

        pltpu.sync_copy(ss_tab, ss_all.at[tid])
        plsc.subcore_barrier()
        pltpu.sync_copy(ss_all.at[:, pl.ds(r0, ROWS_W)], ssg)

        def red_col(j, _):
            acc = ssg[0, pl.ds(j * 16, 16)]
            for t in range(1, NS):
                acc = acc + ssg[t, pl.ds(j * 16, 16)]
            ssred[pl.ds(j * 16, 16)] = acc
            return 0
        lax.fori_loop(0, ROWS_W // 16, red_col, 0)
        pltpu.sync_copy(ssred, ss_hbm.at[cid, pl.ds(r0, ROWS_W)])

    return sc_a(sdt, srcr, dstr, ear)


def _sc_row_pass(ft, src4, dst4, e4):
    mesh = plsc.VectorSubcoreMesh(core_axis_name="c", subcore_axis_name="s")

    @functools.partial(
        pl.kernel,
        mesh=mesh,
        compiler_params=_SC_PARAMS,
        out_type=jax.ShapeDtypeStruct((NC, NP, D), jnp.float32),
        scratch_types=[
            pltpu.VMEM((2, BLK, CHUNK), jnp.int32),
            pltpu.VMEM((2, BLK, CHUNK), jnp.int32),
            pltpu.VMEM((2, BLK, CHUNK), jnp.float32),
            pltpu.VMEM((2, CHUNK, D), jnp.float32),
            pltpu.VMEM_SHARED((NP, D), jnp.float32),
            pltpu.SemaphoreType.DMA,
            pltpu.SemaphoreType.DMA,
            pltpu.SemaphoreType.DMA,
        ],
    )
    def sc_b(ft_hbm, src_hbm, dst_hbm, e_hbm, num_hbm,
             srcb, dstb, eb, rows, num_sh, sem_blk, sem_g, sem_s):
        cid = lax.axis_index("c")
        tid = lax.axis_index("s")
        wid = tid * NC + cid
        r0 = tid * ROWS_W

        zf16 = jnp.zeros((16,), jnp.float32)

        def zrow(j, _):
            for h in range(D // 16):
                rows[0, j, pl.ds(h * 16, 16)] = zf16
            return 0
        lax.fori_loop(0, CHUNK, zrow, 0)
        for i in range(ROWS_W // CHUNK):
            pltpu.sync_copy(rows.at[0], num_sh.at[pl.ds(r0 + i * CHUNK, CHUNK)])

        def fetch_block(b, slot):
            pltpu.async_copy(src_hbm.at[wid, b], srcb.at[slot], sem_blk)
            pltpu.async_copy(dst_hbm.at[wid, b], dstb.at[slot], sem_blk)
            pltpu.async_copy(e_hbm.at[wid, b], eb.at[slot], sem_blk)

        def drain_block(slot):
            pltpu.make_async_copy(src_hbm.at[wid, 0], srcb.at[slot], sem_blk).wait()
            pltpu.make_async_copy(dst_hbm.at[wid, 0], dstb.at[slot], sem_blk).wait()
            pltpu.make_async_copy(e_hbm.at[wid, 0], eb.at[slot], sem_blk).wait()

        fetch_block(0, 0)
        plsc.subcore_barrier()
        drain_block(0)
        fetch_block(1, 1)
        pltpu.async_copy(ft_hbm.at[srcb.at[0, 0]], rows.at[0], sem_g)

        def chunk_body(k, _):
            p = k & 1
            b = k // BLK
            bq = b & 1
            cidx = k - b * BLK
            kn = k + 1
            bn = (kn // BLK) & 1
            cn = kn - (kn // BLK) * BLK

            pltpu.make_async_copy(ft_hbm.at[srcb.at[bq, cidx]], rows.at[p],
                                  sem_g).wait()

            def scale(grp, _):
                evec = eb[bq, cidx, pl.ds(grp * 16, 16)]
                for j in range(16):
                    e = evec[j]
                    r = grp * 16 + j
                    for h in range(D // 16):
                        rows[p, r, pl.ds(h * 16, 16)] = (
                            rows[p, r, pl.ds(h * 16, 16)] * e)
                return 0
            lax.fori_loop(0, CHUNK // 16, scale, 0)

            @pl.when(k >= 1)
            def _():
                pltpu.make_async_copy(rows.at[1 - p],
                                      num_sh.at[dstb.at[bq, cidx]],
                                      sem_s).wait()
            pltpu.async_copy(rows.at[p], num_sh.at[dstb.at[bq, cidx]], sem_s,
                             add=True)

            @pl.when(kn < NCHUNK)
            def _issue_next():
                @pl.when(cn == 0)
                def _():
                    drain_block(bn)
                pltpu.async_copy(ft_hbm.at[srcb.at[bn, cn]], rows.at[1 - p],
                                 sem_g)

            @pl.when(jnp.logical_and(cidx == BLK - 1, b + 2 < NBLK))
            def _prefetch():
                fetch_block(b + 2, bq)
            return 0
        lax.fori_loop(0, NCHUNK, chunk_body, 0)

        pltpu.make_async_copy(rows.at[0], num_sh.at[dstb.at[0, 0]],
                              sem_s).wait()
        plsc.subcore_barrier()
        pltpu.sync_copy(num_sh.at[pl.ds(r0, ROWS_W)],
                        num_hbm.at[cid, pl.ds(r0, ROWS_W)])

    return sc_b(ft, src4, dst4, e4)


def _tc2_body(n0_ref, n1_ref, s0_ref, s1_ref, o_ref):
    s = s0_ref[...] + s1_ref[...] + 1e-16
    v = (n0_ref[...] + n1_ref[...]) / s
    o_ref[...] = jnp.where(v >= 0, v, 0.01 * v)


def _tc2(n0, n1, s0, s1):
    grid = 10
    nb = NP // grid
    return pl.pallas_call(
        _tc2_body,
        grid=(grid,),
        in_specs=[
            pl.BlockSpec((nb, D), lambda i: (i, 0)),
            pl.BlockSpec((nb, D), lambda i: (i, 0)),
            pl.BlockSpec((nb, 1), lambda i: (i, 0)),
            pl.BlockSpec((nb, 1), lambda i: (i, 0)),
        ],
        out_specs=pl.BlockSpec((nb, D), lambda i: (i, 0)),
        out_shape=jax.ShapeDtypeStruct((NP, D), jnp.float32),
    )(n0, n1, s0, s1)


@jax.jit
def kernel(x, edge_index, edge_attr, W, a_w):
    a = a_w[:, 0]
    asd = jnp.stack([a[:D], a[D + DE:]], axis=1)
    ae_blkdiag = jnp.kron(jnp.eye(128, dtype=jnp.float32), a[D:D + DE][:, None])

    ft, sd = _tc1(x, W, asd)
    ea = _tc1e(ae_blkdiag, edge_attr.reshape(E // 128, 128 * DE))

    sdt = jnp.pad(sd, ((0, NP - N), (0, 0))).reshape(2 * NP)
    srcr = edge_index[0].reshape(NW, NCHUNK, CHUNK)
    dstr = edge_index[1].reshape(NW, NCHUNK, CHUNK)
    ear = ea.reshape(NW, NCHUNK, CHUNK)

    ev, ss = _sc_scalar_pass(sdt, srcr, dstr, ear)

    src4 = edge_index[0].reshape(NW, NBLK, BLK, CHUNK)
    dst4 = edge_index[1].reshape(NW, NBLK, BLK, CHUNK)
    e4 = ev.reshape(NW, NBLK, BLK, CHUNK)

    num_p = _sc_row_pass(ft, src4, dst4, e4)

    out = _tc2(num_p[0], num_p[1], ss[0, :, None], ss[1, :, None])
    return out[:N]


# speedup vs baseline: 1.6681x; 1.6681x over previous
import functools

import jax
import jax.numpy as jnp
from jax import lax
from jax.experimental import pallas as pl
from jax.experimental.pallas import tpu as pltpu
from jax.experimental.pallas import tpu_sc as plsc

N = 10000
E = 320000
D = 128
DE = 16

NP = 10240
NC = 2
NS = 16
NW = NC * NS
EW = E // NW
CHUNK = 80
NCHUNK = EW // CHUNK
BLK = 5
NBLK = NCHUNK // BLK
ROWS_W = NP // NS

_SC_PARAMS = pltpu.CompilerParams(
    needs_layout_passes=False, use_tc_tiling_on_sc=False)


def _tc1_body(x_ref, w_ref, asd_ref, ft_ref, sd_ref):
    xb = x_ref[...]
    ft_ref[...] = jnp.dot(xb, w_ref[...], preferred_element_type=jnp.float32)
    sd_ref[...] = jnp.dot(xb, asd_ref[...], preferred_element_type=jnp.float32)


def _tc1(x, W, asd):
    grid = 10
    nb = N // grid
    return pl.pallas_call(
        _tc1_body,
        grid=(grid,),
        in_specs=[
            pl.BlockSpec((nb, D), lambda i: (i, 0)),
            pl.BlockSpec((D, D), lambda i: (0, 0)),
            pl.BlockSpec((D, 2), lambda i: (0, 0)),
        ],
        out_specs=[
            pl.BlockSpec((nb, D), lambda i: (i, 0)),
            pl.BlockSpec((nb, 2), lambda i: (i, 0)),
        ],
        out_shape=[
            jax.ShapeDtypeStruct((N, D), jnp.float32),
            jax.ShapeDtypeStruct((N, 2), jnp.float32),
        ],
    )(x, W, asd)


def _tc1e_body(ae_ref, eattr_ref, ea_ref):
    @pl.when(pl.program_id(0) == 0)
    def _():
        ea_ref[...] = jnp.zeros_like(ea_ref)
    ea_ref[...] += jnp.dot(eattr_ref[...], ae_ref[...],
                           preferred_element_type=jnp.float32)


def _tc1e(ae_blkdiag, edge_attr128):
    grid = 8
    kb = 128 * DE // grid
    return pl.pallas_call(
        _tc1e_body,
        grid=(grid,),
        in_specs=[
            pl.BlockSpec((kb, 128), lambda i: (i, 0)),
            pl.BlockSpec((E // 128, kb), lambda i: (0, i)),
        ],
        out_specs=pl.BlockSpec((E // 128, 128), lambda i: (0, 0)),
        out_shape=jax.ShapeDtypeStruct((E // 128, 128), jnp.float32),
    )(ae_blkdiag, edge_attr128)


def _sc_scalar_pass(sdt, srcr, dstr, ear):
    mesh = plsc.VectorSubcoreMesh(core_axis_name="c", subcore_axis_name="s")

    @functools.partial(
        pl.kernel,
        mesh=mesh,
        compiler_params=_SC_PARAMS,
        out_type=[
            jax.ShapeDtypeStruct((NW, NCHUNK, CHUNK), jnp.float32),
            jax.ShapeDtypeStruct((NC, NP), jnp.float32),
        ],
        scratch_types=[
            pltpu.VMEM((2 * NP,), jnp.float32),
            pltpu.VMEM((NCHUNK, CHUNK), jnp.int32),
            pltpu.VMEM((NCHUNK, CHUNK), jnp.int32),
            pltpu.VMEM((NCHUNK, CHUNK), jnp.float32),
            pltpu.VMEM((NP,), jnp.float32),
            pltpu.VMEM((NS, ROWS_W), jnp.float32),
            pltpu.VMEM((ROWS_W,), jnp.float32),
            pltpu.VMEM_SHARED((NS, NP), jnp.float32),
        ],
    )
    def sc_a(sdt_hbm, src_hbm, dst_hbm, ea_hbm, e_hbm, ss_hbm,
             sdt_v, srcc, dstc, eac, ss_tab, ssg, ssred, ss_all):
        cid = lax.axis_index("c")
        tid = lax.axis_index("s")
        wid = tid * NC + cid
        r0 = tid * ROWS_W

        zf16 = jnp.zeros((16,), jnp.float32)

        pltpu.sync_copy(sdt_hbm, sdt_v)
        pltpu.sync_copy(src_hbm.at[wid], srcc)
        pltpu.sync_copy(dst_hbm.at[wid], dstc)
        pltpu.sync_copy(ea_hbm.at[wid], eac)

        def zss(i, _):
            ss_tab[pl.ds(i * 16, 16)] = zf16
            return 0
        lax.fori_loop(0, NP // 16, zss, 0)

        def edge_chunk(c, _):
            def edge_grp(g, _):
                srcv = srcc[c, pl.ds(g * 16, 16)]
                dstv = dstc[c, pl.ds(g * 16, 16)]
                eav = eac[c, pl.ds(g * 16, 16)]
                s = plsc.load_gather(sdt_v, [srcv * 2])
                dd = plsc.load_gather(sdt_v, [dstv * 2 + 1])
                ev = jnp.exp(s + dd + eav)
                eac[c, pl.ds(g * 16, 16)] = ev
                plsc.addupdate_scatter(ss_tab, [dstv], ev)
                return 0
            lax.fori_loop(0, CHUNK // 16, edge_grp, 0)
            return 0
        lax.fori_loop(0, NCHUNK, edge_chunk, 0)

        pltpu.sync_copy(eac, e_hbm.at[wid])

        pltpu.sync_copy(ss_tab, ss_all.at[tid])
        plsc.subcore_barrier()
        pltpu.sync_copy(ss_all.at[:, pl.ds(r0, ROWS_W)], ssg)

        def red_col(j, _):
            acc = ssg[0, pl.ds(j * 16, 16)]
            for t in range(1, NS):
                acc = acc + ssg[t, pl.ds(j * 16, 16)]
            ssred[pl.ds(j * 16, 16)] = acc
            return 0
        lax.fori_loop(0, ROWS_W // 16, red_col, 0)
        pltpu.sync_copy(ssred, ss_hbm.at[cid, pl.ds(r0, ROWS_W)])

    return sc_a(sdt, srcr, dstr, ear)


def _sc_row_pass(ft, src4, dst4, e4):
    mesh = plsc.VectorSubcoreMesh(core_axis_name="c", subcore_axis_name="s")

    @functools.partial(
        pl.kernel,
        mesh=mesh,
        compiler_params=_SC_PARAMS,
        out_type=jax.ShapeDtypeStruct((NC, NP, D), jnp.float32),
        scratch_types=[
            pltpu.VMEM((2, BLK, CHUNK), jnp.int32),
            pltpu.VMEM((2, BLK, CHUNK), jnp.int32),
            pltpu.VMEM((2, BLK, CHUNK), jnp.float32),
            pltpu.VMEM((2, CHUNK, D), jnp.float32),
            pltpu.VMEM_SHARED((NP, D), jnp.float32),
            pltpu.SemaphoreType.DMA,
            pltpu.SemaphoreType.DMA,
            pltpu.SemaphoreType.DMA,
        ],
    )
    def sc_b(ft_hbm, src_hbm, dst_hbm, e_hbm, num_hbm,
             srcb, dstb, eb, rows, num_sh, sem_blk, sem_g, sem_s):
        cid = lax.axis_index("c")
        tid = lax.axis_index("s")
        wid = tid * NC + cid
        r0 = tid * ROWS_W

        zf16 = jnp.zeros((16,), jnp.float32)

        def zrow(j, _):
            for h in range(D // 16):
                rows[0, j, pl.ds(h * 16, 16)] = zf16
            return 0
        lax.fori_loop(0, CHUNK, zrow, 0)
        for i in range(ROWS_W // CHUNK):
            pltpu.sync_copy(rows.at[0], num_sh.at[pl.ds(r0 + i * CHUNK, CHUNK)])

        def fetch_block(b, slot):
            pltpu.async_copy(src_hbm.at[wid, b], srcb.at[slot], sem_blk)
            pltpu.async_copy(dst_hbm.at[wid, b], dstb.at[slot], sem_blk)
            pltpu.async_copy(e_hbm.at[wid, b], eb.at[slot], sem_blk)

        def drain_block(slot):
            pltpu.make_async_copy(src_hbm.at[wid, 0], srcb.at[slot],
                                  sem_blk).wait()
            pltpu.make_async_copy(dst_hbm.at[wid, 0], dstb.at[slot],
                                  sem_blk).wait()
            pltpu.make_async_copy(e_hbm.at[wid, 0], eb.at[slot],
                                  sem_blk).wait()

        fetch_block(0, 0)
        plsc.subcore_barrier()
        drain_block(0)
        fetch_block(1, 1)
        pltpu.async_copy(ft_hbm.at[srcb.at[0, 0]], rows.at[0], sem_g)

        def process_chunk(k, myrows, otherrows):
            b = k // BLK
            bq = b & 1
            cidx = k - b * BLK
            kn = k + 1
            bn = (kn // BLK) & 1
            cn = kn - (kn // BLK) * BLK

            pltpu.make_async_copy(ft_hbm.at[srcb.at[bq, cidx]], myrows,
                                  sem_g).wait()

            def scale(grp, _):
                evec = eb[bq, cidx, pl.ds(grp * 16, 16)]
                for j in range(16):
                    e = evec[j]
                    r = grp * 16 + j
                    for h in range(D // 16):
                        myrows[r, pl.ds(h * 16, 16)] = (
                            myrows[r, pl.ds(h * 16, 16)] * e)
                return 0
            lax.fori_loop(0, CHUNK // 16, scale, 0)

            @pl.when(k >= 1)
            def _():
                pltpu.make_async_copy(otherrows,
                                      num_sh.at[dstb.at[bq, cidx]],
                                      sem_s).wait()
            pltpu.async_copy(myrows, num_sh.at[dstb.at[bq, cidx]], sem_s,
                             add=True)

            @pl.when(kn < NCHUNK)
            def _issue_next():
                @pl.when(cn == 0)
                def _():
                    drain_block(bn)
                pltpu.async_copy(ft_hbm.at[srcb.at[bn, cn]], otherrows, sem_g)

            @pl.when(jnp.logical_and(cidx == BLK - 1, b + 2 < NBLK))
            def _prefetch():
                fetch_block(b + 2, bq)

        def pair_body(k2, _):
            process_chunk(k2 * 2, rows.at[0], rows.at[1])
            process_chunk(k2 * 2 + 1, rows.at[1], rows.at[0])
            return 0
        lax.fori_loop(0, NCHUNK // 2, pair_body, 0)
        process_chunk(NCHUNK - 1, rows.at[0], rows.at[1])

        pltpu.make_async_copy(rows.at[0], num_sh.at[dstb.at[0, 0]],
                              sem_s).wait()

        plsc.subcore_barrier()
        pltpu.sync_copy(num_sh.at[pl.ds(r0, ROWS_W)],
                        num_hbm.at[cid, pl.ds(r0, ROWS_W)])

    return sc_b(ft, src4, dst4, e4)


def _tc2_body(n0_ref, n1_ref, s0_ref, s1_ref, o_ref):
    s = s0_ref[...] + s1_ref[...] + 1e-16
    v = (n0_ref[...] + n1_ref[...]) / s
    o_ref[...] = jnp.where(v >= 0, v, 0.01 * v)


def _tc2(n0, n1, s0, s1):
    grid = 10
    nb = NP // grid
    return pl.pallas_call(
        _tc2_body,
        grid=(grid,),
        in_specs=[
            pl.BlockSpec((nb, D), lambda i: (i, 0)),
            pl.BlockSpec((nb, D), lambda i: (i, 0)),
            pl.BlockSpec((nb, 1), lambda i: (i, 0)),
            pl.BlockSpec((nb, 1), lambda i: (i, 0)),
        ],
        out_specs=pl.BlockSpec((nb, D), lambda i: (i, 0)),
        out_shape=jax.ShapeDtypeStruct((NP, D), jnp.float32),
    )(n0, n1, s0, s1)


@jax.jit
def kernel(x, edge_index, edge_attr, W, a_w):
    a = a_w[:, 0]
    asd = jnp.stack([a[:D], a[D + DE:]], axis=1)
    ae_blkdiag = jnp.kron(jnp.eye(128, dtype=jnp.float32), a[D:D + DE][:, None])

    ft, sd = _tc1(x, W, asd)
    ea = _tc1e(ae_blkdiag, edge_attr.reshape(E // 128, 128 * DE))

    sdt = jnp.pad(sd, ((0, NP - N), (0, 0))).reshape(2 * NP)
    srcr = edge_index[0].reshape(NW, NCHUNK, CHUNK)
    dstr = edge_index[1].reshape(NW, NCHUNK, CHUNK)
    ear = ea.reshape(NW, NCHUNK, CHUNK)

    ev, ss = _sc_scalar_pass(sdt, srcr, dstr, ear)

    src4 = edge_index[0].reshape(NW, NBLK, BLK, CHUNK)
    dst4 = edge_index[1].reshape(NW, NBLK, BLK, CHUNK)
    e4 = ev.reshape(NW, NBLK, BLK, CHUNK)

    num_p = _sc_row_pass(ft, src4, dst4, e4)

    out = _tc2(num_p[0], num_p[1], ss[0, :, None], ss[1, :, None])
    return out[:N]


# speedup vs baseline: 1.7234x; 1.0331x over previous
import functools

import jax
import jax.numpy as jnp
from jax import lax
from jax.experimental import pallas as pl
from jax.experimental.pallas import tpu as pltpu
from jax.experimental.pallas import tpu_sc as plsc

N = 10000
E = 320000
D = 128
DE = 16

NP = 10240
NC = 2
NS = 16
NW = NC * NS
EW = E // NW
CHUNK = 80
NCHUNK = EW // CHUNK
BLK = 5
NBLK = NCHUNK // BLK
ROWS_W = NP // NS

_SC_PARAMS = pltpu.CompilerParams(
    needs_layout_passes=False, use_tc_tiling_on_sc=False)


def _tc1_body(x_ref, w_ref, asd_ref, ft_ref, sd_ref):
    xb = x_ref[...]
    ft_ref[...] = jnp.dot(xb, w_ref[...], preferred_element_type=jnp.float32)
    sd_ref[...] = jnp.dot(xb, asd_ref[...], preferred_element_type=jnp.float32)


def _tc1(x, W, asd):
    grid = 10
    nb = N // grid
    return pl.pallas_call(
        _tc1_body,
        grid=(grid,),
        in_specs=[
            pl.BlockSpec((nb, D), lambda i: (i, 0)),
            pl.BlockSpec((D, D), lambda i: (0, 0)),
            pl.BlockSpec((D, 2), lambda i: (0, 0)),
        ],
        out_specs=[
            pl.BlockSpec((nb, D), lambda i: (i, 0)),
            pl.BlockSpec((nb, 2), lambda i: (i, 0)),
        ],
        out_shape=[
            jax.ShapeDtypeStruct((N, D), jnp.float32),
            jax.ShapeDtypeStruct((N, 2), jnp.float32),
        ],
    )(x, W, asd)


def _tc1e_body(ae_ref, eattr_ref, ea_ref):
    i = pl.program_id(0)
    t = jnp.dot(eattr_ref[...], ae_ref[...],
                preferred_element_type=jnp.float32)
    ea_ref[pl.ds(i * (t.shape[0] // 128), t.shape[0] // 128), :] = (
        t.reshape(t.shape[0] // 128, 128))


def _tc1e(ae_col, edge_attr):
    grid = 10
    eb = E // grid
    return pl.pallas_call(
        _tc1e_body,
        grid=(grid,),
        in_specs=[
            pl.BlockSpec((DE, 1), lambda i: (0, 0)),
            pl.BlockSpec((eb, DE), lambda i: (i, 0)),
        ],
        out_specs=pl.BlockSpec((E // 128, 128), lambda i: (0, 0)),
        out_shape=jax.ShapeDtypeStruct((E // 128, 128), jnp.float32),
    )(ae_col, edge_attr)


def _sc_scalar_pass(sdt, srcr, dstr, ear):
    mesh = plsc.VectorSubcoreMesh(core_axis_name="c", subcore_axis_name="s")

    @functools.partial(
        pl.kernel,
        mesh=mesh,
        compiler_params=_SC_PARAMS,
        out_type=[
            jax.ShapeDtypeStruct((NW, NCHUNK, CHUNK), jnp.float32),
            jax.ShapeDtypeStruct((NC, NP), jnp.float32),
        ],
        scratch_types=[
            pltpu.VMEM((2 * NP,), jnp.float32),
            pltpu.VMEM((NCHUNK, CHUNK), jnp.int32),
            pltpu.VMEM((NCHUNK, CHUNK), jnp.int32),
            pltpu.VMEM((NCHUNK, CHUNK), jnp.float32),
            pltpu.VMEM((NP,), jnp.float32),
            pltpu.VMEM((NS, ROWS_W), jnp.float32),
            pltpu.VMEM((ROWS_W,), jnp.float32),
            pltpu.VMEM_SHARED((NS, NP), jnp.float32),
        ],
    )
    def sc_a(sdt_hbm, src_hbm, dst_hbm, ea_hbm, e_hbm, ss_hbm,
             sdt_v, srcc, dstc, eac, ss_tab, ssg, ssred, ss_all):
        cid = lax.axis_index("c")
        tid = lax.axis_index("s")
        wid = tid * NC + cid
        r0 = tid * ROWS_W

        zf16 = jnp.zeros((16,), jnp.float32)

        pltpu.sync_copy(sdt_hbm, sdt_v)
        pltpu.sync_copy(src_hbm.at[wid], srcc)
        pltpu.sync_copy(dst_hbm.at[wid], dstc)
        pltpu.sync_copy(ea_hbm.at[wid], eac)

        def zss(i, _):
            ss_tab[pl.ds(i * 16, 16)] = zf16
            return 0
        lax.fori_loop(0, NP // 16, zss, 0)

        def edge_chunk(c, _):
            def edge_grp(g, _):
                srcv = srcc[c, pl.ds(g * 16, 16)]
                dstv = dstc[c, pl.ds(g * 16, 16)]
                eav = eac[c, pl.ds(g * 16, 16)]
                s = plsc.load_gather(sdt_v, [srcv * 2])
                dd = plsc.load_gather(sdt_v, [dstv * 2 + 1])
                ev = jnp.exp(s + dd + eav)
                eac[c, pl.ds(g * 16, 16)] = ev
                plsc.addupdate_scatter(ss_tab, [dstv], ev)
                return 0
            lax.fori_loop(0, CHUNK // 16, edge_grp, 0)
            return 0
        lax.fori_loop(0, NCHUNK, edge_chunk, 0)

        pltpu.sync_copy(eac, e_hbm.at[wid])

        pltpu.sync_copy(ss_tab, ss_all.at[tid])
        plsc.subcore_barrier()
        pltpu.sync_copy(ss_all.at[:, pl.ds(r0, ROWS_W)], ssg)

        def red_col(j, _):
            acc = ssg[0, pl.ds(j * 16, 16)]
            for t in range(1, NS):
                acc = acc + ssg[t, pl.ds(j * 16, 16)]
            ssred[pl.ds(j * 16, 16)] = acc
            return 0
        lax.fori_loop(0, ROWS_W // 16, red_col, 0)
        pltpu.sync_copy(ssred, ss_hbm.at[cid, pl.ds(r0, ROWS_W)])

    return sc_a(sdt, srcr, dstr, ear)


def _sc_row_pass(ft, src4, dst4, e4):
    mesh = plsc.VectorSubcoreMesh(core_axis_name="c", subcore_axis_name="s")

    @functools.partial(
        pl.kernel,
        mesh=mesh,
        compiler_params=_SC_PARAMS,
        out_type=jax.ShapeDtypeStruct((NC, NP, D), jnp.float32),
        scratch_types=[
            pltpu.VMEM((2, BLK, CHUNK), jnp.int32),
            pltpu.VMEM((2, BLK, CHUNK), jnp.int32),
            pltpu.VMEM((2, BLK, CHUNK), jnp.float32),
            pltpu.VMEM((2, CHUNK, D), jnp.float32),
            pltpu.VMEM_SHARED((NP, D), jnp.float32),
            pltpu.SemaphoreType.DMA,
            pltpu.SemaphoreType.DMA,
            pltpu.SemaphoreType.DMA,
        ],
    )
    def sc_b(ft_hbm, src_hbm, dst_hbm, e_hbm, num_hbm,
             srcb, dstb, eb, rows, num_sh, sem_blk, sem_g, sem_s):
        cid = lax.axis_index("c")
        tid = lax.axis_index("s")
        wid = tid * NC + cid
        r0 = tid * ROWS_W

        zf16 = jnp.zeros((16,), jnp.float32)

        def zrow(j, _):
            for h in range(D // 16):
                rows[0, j, pl.ds(h * 16, 16)] = zf16
            return 0
        lax.fori_loop(0, CHUNK, zrow, 0)
        for i in range(ROWS_W // CHUNK):
            pltpu.sync_copy(rows.at[0], num_sh.at[pl.ds(r0 + i * CHUNK, CHUNK)])

        def fetch_block(b, slot):
            pltpu.async_copy(src_hbm.at[wid, b], srcb.at[slot], sem_blk)
            pltpu.async_copy(dst_hbm.at[wid, b], dstb.at[slot], sem_blk)
            pltpu.async_copy(e_hbm.at[wid, b], eb.at[slot], sem_blk)

        def drain_block(slot):
            pltpu.make_async_copy(src_hbm.at[wid, 0], srcb.at[slot],
                                  sem_blk).wait()
            pltpu.make_async_copy(dst_hbm.at[wid, 0], dstb.at[slot],
                                  sem_blk).wait()
            pltpu.make_async_copy(e_hbm.at[wid, 0], eb.at[slot],
                                  sem_blk).wait()

        fetch_block(0, 0)
        plsc.subcore_barrier()
        drain_block(0)
        fetch_block(1, 1)
        pltpu.async_copy(ft_hbm.at[srcb.at[0, 0]], rows.at[0], sem_g)

        def process_chunk(k, myrows, otherrows):
            b = k // BLK
            bq = b & 1
            cidx = k - b * BLK
            kn = k + 1
            bn = (kn // BLK) & 1
            cn = kn - (kn // BLK) * BLK

            pltpu.make_async_copy(ft_hbm.at[srcb.at[bq, cidx]], myrows,
                                  sem_g).wait()

            def scale(grp, _):
                evec = eb[bq, cidx, pl.ds(grp * 16, 16)]
                for j in range(16):
                    e = evec[j]
                    r = grp * 16 + j
                    for h in range(D // 16):
                        myrows[r, pl.ds(h * 16, 16)] = (
                            myrows[r, pl.ds(h * 16, 16)] * e)
                return 0
            lax.fori_loop(0, CHUNK // 16, scale, 0)

            @pl.when(k >= 1)
            def _():
                pltpu.make_async_copy(otherrows,
                                      num_sh.at[dstb.at[bq, cidx]],
                                      sem_s).wait()
            pltpu.async_copy(myrows, num_sh.at[dstb.at[bq, cidx]], sem_s,
                             add=True)

            @pl.when(kn < NCHUNK)
            def _issue_next():
                @pl.when(cn == 0)
                def _():
                    drain_block(bn)
                pltpu.async_copy(ft_hbm.at[srcb.at[bn, cn]], otherrows, sem_g)

            @pl.when(jnp.logical_and(cidx == BLK - 1, b + 2 < NBLK))
            def _prefetch():
                fetch_block(b + 2, bq)

        def pair_body(k2, _):
            process_chunk(k2 * 2, rows.at[0], rows.at[1])
            process_chunk(k2 * 2 + 1, rows.at[1], rows.at[0])
            return 0
        lax.fori_loop(0, NCHUNK // 2, pair_body, 0)
        process_chunk(NCHUNK - 1, rows.at[0], rows.at[1])

        pltpu.make_async_copy(rows.at[0], num_sh.at[dstb.at[0, 0]],
                              sem_s).wait()

        plsc.subcore_barrier()
        pltpu.sync_copy(num_sh.at[pl.ds(r0, ROWS_W)],
                        num_hbm.at[cid, pl.ds(r0, ROWS_W)])

    return sc_b(ft, src4, dst4, e4)


def _tc2_body(n0_ref, n1_ref, s0_ref, s1_ref, o_ref):
    s = s0_ref[...] + s1_ref[...] + 1e-16
    v = (n0_ref[...] + n1_ref[...]) / s
    o_ref[...] = jnp.where(v >= 0, v, 0.01 * v)


def _tc2(n0, n1, s0, s1):
    grid = 10
    nb = NP // grid
    return pl.pallas_call(
        _tc2_body,
        grid=(grid,),
        in_specs=[
            pl.BlockSpec((nb, D), lambda i: (i, 0)),
            pl.BlockSpec((nb, D), lambda i: (i, 0)),
            pl.BlockSpec((nb, 1), lambda i: (i, 0)),
            pl.BlockSpec((nb, 1), lambda i: (i, 0)),
        ],
        out_specs=pl.BlockSpec((nb, D), lambda i: (i, 0)),
        out_shape=jax.ShapeDtypeStruct((NP, D), jnp.float32),
    )(n0, n1, s0, s1)


@jax.jit
def kernel(x, edge_index, edge_attr, W, a_w):
    a = a_w[:, 0]
    asd = jnp.stack([a[:D], a[D + DE:]], axis=1)
    ft, sd = _tc1(x, W, asd)
    ea = _tc1e(a[D:D + DE][:, None], edge_attr)

    sdt = jnp.pad(sd, ((0, NP - N), (0, 0))).reshape(2 * NP)
    srcr = edge_index[0].reshape(NW, NCHUNK, CHUNK)
    dstr = edge_index[1].reshape(NW, NCHUNK, CHUNK)
    ear = ea.reshape(NW, NCHUNK, CHUNK)

    ev, ss = _sc_scalar_pass(sdt, srcr, dstr, ear)

    src4 = edge_index[0].reshape(NW, NBLK, BLK, CHUNK)
    dst4 = edge_index[1].reshape(NW, NBLK, BLK, CHUNK)
    e4 = ev.reshape(NW, NBLK, BLK, CHUNK)

    num_p = _sc_row_pass(ft, src4, dst4, e4)

    out = _tc2(num_p[0], num_p[1], ss[0, :, None], ss[1, :, None])
    return out[:N]
